# Initial kernel scaffold; baseline (speedup 1.0000x reference)
import jax
import jax.numpy as jnp
from jax.experimental import pallas as pl


def kernel(h, graph_ids, fc_w, fc_b, cls_w, cls_b):
    raise NotImplementedError("write your pallas kernel here")


# speedup vs baseline: 2.2107x; 2.2107x over previous
import jax
import jax.numpy as jnp
from jax import lax
from jax.experimental import pallas as pl
from jax.experimental.pallas import tpu as pltpu
from jax.experimental.pallas import tpu_sc as plsc

N_ROWS = 100000
D = 256
DH = 128
N_SEG = 1024
NC = 2
NS = 16
NW = NC * NS
CHUNK = 128
ACC_ROWS = N_SEG + 128
DUMMY = N_SEG
UNITS = N_ROWS // 16
UNITS_LO = UNITS // NW
UNITS_EXTRA = UNITS - UNITS_LO * NW
N_CHUNKS = ((UNITS_LO + 1) * 16 + CHUNK - 1) // CHUNK
ZROWS = ACC_ROWS // NS
STRIPE = N_SEG // NS


def _seg_body(h3_hbm, ids_hbm, sum_out, cnt_out,
              lo_v, hi_v, ids_v, ones_v, zacc_v,
              acc_lo, acc_hi, cnt_s):
  cid = lax.axis_index("c")
  sid = lax.axis_index("s")
  wid = sid * NC + cid

  zeros16 = jnp.zeros((16,), jnp.float32)
  ones16 = jnp.ones((16,), jnp.float32)
  lane = lax.iota(jnp.int32, 16)

  def _zrow(r, carry):
    for c in range(DH // 16):
      zacc_v[r, pl.ds(c * 16, 16)] = zeros16
    return carry

  lax.fori_loop(0, ZROWS, _zrow, 0)

  def _orow(r, carry):
    for c in range(DH // 16):
      ones_v[r, pl.ds(c * 16, 16)] = ones16
    return carry

  lax.fori_loop(0, CHUNK, _orow, 0)

  pltpu.sync_copy(zacc_v, acc_lo.at[pl.ds(sid * ZROWS, ZROWS)])
  pltpu.sync_copy(zacc_v, acc_hi.at[pl.ds(sid * ZROWS, ZROWS)])
  pltpu.sync_copy(zacc_v, cnt_s.at[pl.ds(sid * ZROWS, ZROWS)])
  plsc.subcore_barrier()

  base = (UNITS_LO * wid + jnp.minimum(wid, UNITS_EXTRA)) * 16
  nrows = (UNITS_LO + jnp.where(wid < UNITS_EXTRA, 1, 0)) * 16
  rend = base + nrows

  def _chunk(i, carry):
    start = base + i * CHUNK
    r0 = jnp.minimum(start, N_ROWS - CHUNK)
    pltpu.sync_copy(ids_hbm.at[pl.ds(r0, CHUNK)], ids_v.at[0])
    pltpu.sync_copy(h3_hbm.at[pl.ds(r0, CHUNK), 0], lo_v)
    pltpu.sync_copy(h3_hbm.at[pl.ds(r0, CHUNK), 1], hi_v)
    for g in range(CHUNK // 16):
      pos = r0 + g * 16 + lane
      v = ids_v[0, pl.ds(g * 16, 16)]
      ok = (pos >= start) & (pos < rend)
      ids_v[0, pl.ds(g * 16, 16)] = jnp.where(ok, v, DUMMY)
    pltpu.sync_copy(lo_v, acc_lo.at[ids_v.at[0]], add=True)
    pltpu.sync_copy(hi_v, acc_hi.at[ids_v.at[0]], add=True)
    pltpu.sync_copy(ones_v, cnt_s.at[ids_v.at[0]], add=True)
    return carry

  lax.fori_loop(0, N_CHUNKS, _chunk, 0)
  plsc.subcore_barrier()

  ob = sid * STRIPE
  pltpu.sync_copy(acc_lo.at[pl.ds(ob, STRIPE)],
                  sum_out.at[cid, 0, pl.ds(ob, STRIPE)])
  pltpu.sync_copy(acc_hi.at[pl.ds(ob, STRIPE)],
                  sum_out.at[cid, 1, pl.ds(ob, STRIPE)])
  pltpu.sync_copy(cnt_s.at[pl.ds(ob, STRIPE)],
                  cnt_out.at[cid, pl.ds(ob, STRIPE)])


_seg_call = pl.kernel(
    _seg_body,
    out_type=(
        jax.ShapeDtypeStruct((NC, 2, N_SEG, DH), jnp.float32),
        jax.ShapeDtypeStruct((NC, N_SEG, DH), jnp.float32),
    ),
    mesh=plsc.VectorSubcoreMesh(core_axis_name="c", subcore_axis_name="s",
                                num_cores=NC, num_subcores=NS),
    scratch_types=[
        pltpu.VMEM((CHUNK, DH), jnp.float32),
        pltpu.VMEM((CHUNK, DH), jnp.float32),
        pltpu.VMEM((1, CHUNK), jnp.int32),
        pltpu.VMEM((CHUNK, DH), jnp.float32),
        pltpu.VMEM((ZROWS, DH), jnp.float32),
        pltpu.VMEM_SHARED((ACC_ROWS, DH), jnp.float32),
        pltpu.VMEM_SHARED((ACC_ROWS, DH), jnp.float32),
        pltpu.VMEM_SHARED((ACC_ROWS, DH), jnp.float32),
    ],
)


def _mlp_body(sum_ref, cnt_ref, fcw_ref, fcb_ref, clsw_ref, clsb_ref,
              out_ref):
  s4 = sum_ref[...]
  c3 = cnt_ref[...]
  s = jnp.concatenate([s4[0, 0] + s4[1, 0], s4[0, 1] + s4[1, 1]], axis=1)
  c = c3[0, :, 0:1] + c3[1, :, 0:1]
  feats = s / jnp.maximum(c, 1.0)
  hidden = jnp.dot(feats, fcw_ref[...],
                   preferred_element_type=jnp.float32) + fcb_ref[...]
  hidden = jnp.maximum(hidden, 0.0)
  out_ref[...] = jnp.dot(hidden, clsw_ref[...],
                         preferred_element_type=jnp.float32) + clsb_ref[...]


def kernel(h, graph_ids, fc_w, fc_b, cls_w, cls_b):
  ids = graph_ids.astype(jnp.int32)
  h3 = h.reshape(N_ROWS, 2, DH)

  sums, cnts = _seg_call(h3, ids)

  out = pl.pallas_call(
      _mlp_body,
      out_shape=jax.ShapeDtypeStruct((N_SEG, 16), jnp.float32),
  )(sums, cnts, fc_w, fc_b.reshape(1, -1), cls_w, cls_b.reshape(1, -1))
  return out
